# baseline (speedup 1.0000x reference)
import math

import jax
import jax.numpy as jnp
from jax import lax
from jax.experimental import pallas as pl
from jax.experimental.pallas import tpu as pltpu
from jax.experimental.pallas import tpu_sc as plsc

_VOCAB = 1000000
_D = 64
_B = 4096
_S = 200

_NC = 2
_NS = 16
_NW = _NC * _NS

_ROWS = _B * _S
_RPW = _ROWS // _NW
_CH = _S
_NCH = _RPW // _CH
_HALF = _CH // 2
_XROWS = _ROWS // _HALF
_BPW = _B // _NW


def _pe_table():
    pos = jnp.arange(0, _S, dtype=jnp.float32)[:, None]
    ang = pos * jnp.exp(
        -jnp.arange(0, _D, 2, dtype=jnp.float32) * math.log(1000.0) / _D)
    pe = jnp.zeros((_S, _D), dtype=jnp.float32)
    pe = pe.at[:, 0::2].set(jnp.sin(ang))
    pe = pe.at[:, 1::2].set(jnp.cos(ang))
    return pe


def _body(x2, W, pe, out, pe_v, idx_v, obuf,
          sem_i0, sem_i1, sem_i2, sem_i3, sem_g0, sem_g1, sem_o0, sem_o1):
    sem_i = ((sem_i0, sem_i1), (sem_i2, sem_i3))
    sem_g = (sem_g0, sem_g1)
    sem_o = (sem_o0, sem_o1)
    cid = lax.axis_index("c")
    sid = lax.axis_index("s")
    wid = sid * _NC + cid
    xbase = wid * (_RPW // _HALF)
    obase = wid * _BPW

    pltpu.sync_copy(pe, pe_v)

    def fire_idx(b, j, cc):
        pltpu.async_copy(x2.at[pl.ds(xbase + cc * 2, 2)], idx_v.at[b, j],
                         sem_i[b][j])

    def wait_idx(b, j):
        pltpu.make_async_copy(x2.at[pl.ds(xbase, 2)], idx_v.at[b, j],
                              sem_i[b][j]).wait()

    def fire_gathers(b, j):
        for q in range(2):
            pltpu.async_copy(W.at[idx_v.at[b, j, q]],
                             obuf.at[b, pl.ds(q * _HALF, _HALF)],
                             sem_g[b], add=True)

    def wait_gathers(b, j):
        for q in range(2):
            pltpu.make_async_copy(W.at[idx_v.at[b, j, q]],
                                  obuf.at[b, pl.ds(q * _HALF, _HALF)],
                                  sem_g[b]).wait()

    def fire_store(b, cc):
        pltpu.async_copy(obuf.at[b], out.at[obase + cc], sem_o[b])

    def prefill_pe(b):
        @pl.loop(0, _S, unroll=4)
        def _(r):
            for kk in range(_D // 16):
                sl = pl.ds(kk * 16, 16)
                obuf[b, r, sl] = pe_v[r, sl]

    def wait_store(b):
        pltpu.make_async_copy(obuf.at[b], out.at[obase], sem_o[b]).wait()

    for k in range(4):
        fire_idx(k & 1, (k >> 1) & 1, k)

    @pl.loop(0, _NCH, step=4)
    def _(c):
        for k in range(4):
            b = k & 1
            j = (k >> 1) & 1
            cc = c + k
            pb = 1 - b
            pj = j ^ (1 - b)

            def retire_prev():
                wait_gathers(pb, pj)
                @pl.when(cc + 3 < _NCH)
                def _():
                    fire_idx(pb, pj, cc + 3)
                fire_store(pb, cc - 1)

            if k == 0:
                @pl.when(c > 0)
                def _():
                    retire_prev()
            else:
                retire_prev()

            if k >= 2:
                wait_store(b)
            else:
                @pl.when(c > 0)
                def _():
                    wait_store(b)

            prefill_pe(b)
            wait_idx(b, j)
            fire_gathers(b, j)

    wait_gathers(1, 1)
    fire_store(1, _NCH - 1)
    wait_store(0)
    wait_store(1)


def kernel(x, W):
    pe = _pe_table()
    x2 = x.astype(jnp.int32).reshape(_XROWS, _HALF)
    call = pl.kernel(
        _body,
        out_type=jax.ShapeDtypeStruct((_B, _S, _D), jnp.float32),
        mesh=plsc.VectorSubcoreMesh(core_axis_name="c", subcore_axis_name="s"),
        compiler_params=pltpu.CompilerParams(use_tc_tiling_on_sc=False),
        scratch_types=[
            pltpu.VMEM((_S, _D), jnp.float32),
            pltpu.VMEM((2, 2, 2, _HALF), jnp.int32),
            pltpu.VMEM((2, _S, _D), jnp.float32),
            pltpu.SemaphoreType.DMA,
            pltpu.SemaphoreType.DMA,
            pltpu.SemaphoreType.DMA,
            pltpu.SemaphoreType.DMA,
            pltpu.SemaphoreType.DMA,
            pltpu.SemaphoreType.DMA,
            pltpu.SemaphoreType.DMA,
            pltpu.SemaphoreType.DMA,
        ],
    )
    return call(x2, W, pe)
